# Initial kernel scaffold; baseline (speedup 1.0000x reference)
import jax
import jax.numpy as jnp
from jax.experimental import pallas as pl


def kernel(x, edge_index, edge_weight, W_conv, b_conv, W_fc, b_fc, W_fc2, b_fc2):
    raise NotImplementedError("write your pallas kernel here")


# speedup vs baseline: 4.7015x; 4.7015x over previous
import functools

import jax
import jax.numpy as jnp
from jax import lax
from jax.experimental import pallas as pl
from jax.experimental.pallas import tpu as pltpu
from jax.experimental.pallas import tpu_sc as plsc

N_NODES = 10000
N_EDGES = 320000
D = 128
N_PAD = 10240
E_ROWS = N_EDGES // 128
NC = 2
NS = 16
NW = NC * NS
ROWS_PER_TILE = N_PAD // NS

_sc_mesh = plsc.VectorSubcoreMesh(core_axis_name="c", subcore_axis_name="s")
_sc_params = pltpu.CompilerParams(needs_layout_passes=False)


@functools.partial(
    pl.kernel,
    out_type=jax.ShapeDtypeStruct((2 * NW, N_PAD), jnp.float32),
    mesh=_sc_mesh,
    compiler_params=_sc_params,
    scratch_types=[
        pltpu.VMEM((128,), jnp.int32),
        pltpu.VMEM((128,), jnp.int32),
        pltpu.VMEM((N_PAD,), jnp.float32),
        pltpu.VMEM((N_PAD,), jnp.float32),
    ],
)
def _degree_kernel(src_hbm, dst_hbm, out_hbm, src_v, dst_v, dego_v, degi_v):
    cid = lax.axis_index("c")
    sid = lax.axis_index("s")
    wid = sid * NC + cid

    zeros16 = jnp.zeros((16,), jnp.float32)

    def zero_body(k, _):
        dego_v[pl.ds(k * 16, 16)] = zeros16
        degi_v[pl.ds(k * 16, 16)] = zeros16
        return 0

    lax.fori_loop(0, N_PAD // 16, zero_body, 0)

    ones16 = jnp.ones((16,), jnp.float32)
    n_rows = (E_ROWS - wid + NW - 1) // NW

    def row_body(i, _):
        r = wid + i * NW
        pltpu.sync_copy(src_hbm.at[r], src_v)
        pltpu.sync_copy(dst_hbm.at[r], dst_v)
        for j in range(8):
            s16 = src_v[pl.ds(j * 16, 16)]
            plsc.addupdate_scatter(dego_v, [s16], ones16)
            d16 = dst_v[pl.ds(j * 16, 16)]
            plsc.addupdate_scatter(degi_v, [d16], ones16)
        return 0

    lax.fori_loop(0, n_rows, row_body, 0)

    pltpu.sync_copy(dego_v, out_hbm.at[wid])
    pltpu.sync_copy(degi_v, out_hbm.at[NW + wid])


def _norm_body(deg_ref, out_ref):
    d = deg_ref[...]
    deg_out = jnp.sum(d[:NW, :], axis=0, keepdims=True)
    deg_in = jnp.sum(d[NW:, :], axis=0, keepdims=True)
    s = jnp.concatenate([deg_out, deg_in], axis=0)
    out_ref[...] = lax.rsqrt(jnp.maximum(s, 1.0))


def _norms(deg_partial):
    return pl.pallas_call(
        _norm_body,
        out_shape=jax.ShapeDtypeStruct((2, N_PAD), jnp.float32),
    )(deg_partial)


@functools.partial(
    pl.kernel,
    out_type=(
        jax.ShapeDtypeStruct((N_PAD, D), jnp.float32),
        jax.ShapeDtypeStruct((N_PAD, D), jnp.float32),
    ),
    mesh=_sc_mesh,
    compiler_params=_sc_params,
    scratch_types=[
        pltpu.VMEM((N_PAD,), jnp.float32),
        pltpu.VMEM((128,), jnp.int32),
        pltpu.VMEM((128,), jnp.int32),
        pltpu.VMEM((128,), jnp.float32),
        pltpu.VMEM((128,), jnp.float32),
        pltpu.VMEM((128, D), jnp.float32),
        pltpu.VMEM_SHARED((N_PAD, D), jnp.float32),
        pltpu.SemaphoreType.DMA,
    ],
)
def _agg_kernel(x_hbm, src_hbm, dst_hbm, w_hbm, norm_hbm,
                out0_hbm, out1_hbm, norm_v, src_v, dst_v, w_v, wp_v,
                rows_v, agg_sh, sem):
    cid = lax.axis_index("c")
    sid = lax.axis_index("s")
    wid = sid * NC + cid

    zeros16 = jnp.zeros((16,), jnp.float32)

    def zero_body(k, _):
        rows_v[k // 8, pl.ds((k % 8) * 16, 16)] = zeros16
        return 0

    lax.fori_loop(0, 128 * 8, zero_body, 0)
    for b in range(ROWS_PER_TILE // 128):
        pltpu.sync_copy(rows_v, agg_sh.at[pl.ds(sid * ROWS_PER_TILE + b * 128, 128)])
    pltpu.sync_copy(norm_hbm, norm_v)
    plsc.subcore_barrier()

    n_rows = (E_ROWS - wid + NW - 1) // NW

    def row_body(i, _):
        r = wid + i * NW
        pltpu.sync_copy(src_hbm.at[r], src_v)
        pltpu.sync_copy(dst_hbm.at[r], dst_v)
        pltpu.sync_copy(w_hbm.at[r], w_v)
        pltpu.async_copy(x_hbm.at[src_v], rows_v, sem).wait()
        for j in range(8):
            idx16 = src_v[pl.ds(j * 16, 16)]
            nrm16 = plsc.load_gather(norm_v, [idx16])
            wp_v[pl.ds(j * 16, 16)] = w_v[pl.ds(j * 16, 16)] * nrm16

        def edge_body(e, _):
            wsc = plsc.load_gather(wp_v, [jnp.full((16,), e, jnp.int32)])
            for f in range(8):
                rows_v[e, pl.ds(f * 16, 16)] = rows_v[e, pl.ds(f * 16, 16)] * wsc
            return 0

        lax.fori_loop(0, 128, edge_body, 0)
        pltpu.sync_copy(rows_v, agg_sh.at[dst_v], add=True)
        return 0

    lax.fori_loop(0, n_rows, row_body, 0)
    plsc.subcore_barrier()

    @pl.when(cid == 0)
    def _():
        for b in range(ROWS_PER_TILE // 128):
            off = sid * ROWS_PER_TILE + b * 128
            pltpu.sync_copy(agg_sh.at[pl.ds(off, 128)], out0_hbm.at[pl.ds(off, 128)])

    @pl.when(cid == 1)
    def _():
        for b in range(ROWS_PER_TILE // 128):
            off = sid * ROWS_PER_TILE + b * 128
            pltpu.sync_copy(agg_sh.at[pl.ds(off, 128)], out1_hbm.at[pl.ds(off, 128)])


def _mlp_body(a0_ref, a1_ref, nd_ref, wc_ref, bc_ref, wf_ref, bf_ref,
              w2_ref, b2_ref, out_ref):
    h = (a0_ref[...] + a1_ref[...]) * nd_ref[...]
    h = jnp.dot(h, wc_ref[...], preferred_element_type=jnp.float32) + bc_ref[...]
    h = jnp.maximum(h, 0.0)
    h = jnp.dot(h, wf_ref[...], preferred_element_type=jnp.float32) + bf_ref[...]
    h = jnp.maximum(h, 0.0)
    out_ref[...] = (
        jnp.dot(h, w2_ref[...], preferred_element_type=jnp.float32) + b2_ref[...]
    )


def _mlp(a0, a1, norm_dst, W_conv, b_conv, W_fc, b_fc, W_fc2, b_fc2):
    BR = 1000
    grid = (N_NODES // BR,)
    row_spec = pl.BlockSpec((BR, D), lambda i: (i, 0))
    nd_spec = pl.BlockSpec((BR, 1), lambda i: (i, 0))
    w_spec = pl.BlockSpec((D, D), lambda i: (0, 0))
    b_spec = pl.BlockSpec((1, D), lambda i: (0, 0))
    return pl.pallas_call(
        _mlp_body,
        grid=grid,
        in_specs=[row_spec, row_spec, nd_spec, w_spec, b_spec, w_spec,
                  b_spec, w_spec, b_spec],
        out_specs=row_spec,
        out_shape=jax.ShapeDtypeStruct((N_NODES, D), jnp.float32),
    )(a0, a1, norm_dst, W_conv, b_conv, W_fc, b_fc, W_fc2, b_fc2)


@jax.jit
def kernel(x, edge_index, edge_weight, W_conv, b_conv, W_fc, b_fc, W_fc2,
           b_fc2):
    src2d = edge_index[0].astype(jnp.int32).reshape(E_ROWS, 128)
    dst2d = edge_index[1].astype(jnp.int32).reshape(E_ROWS, 128)
    w2d = edge_weight.reshape(E_ROWS, 128)

    deg_partial = _degree_kernel(src2d, dst2d)
    norms = _norms(deg_partial)
    agg0, agg1 = _agg_kernel(x, src2d, dst2d, w2d, norms[0])
    return _mlp(agg0[:N_NODES], agg1[:N_NODES],
                norms[1, :N_NODES, None], W_conv,
                b_conv.reshape(1, D), W_fc, b_fc.reshape(1, D), W_fc2,
                b_fc2.reshape(1, D))


# speedup vs baseline: 4.7943x; 1.0197x over previous
import functools

import jax
import jax.numpy as jnp
from jax import lax
from jax.experimental import pallas as pl
from jax.experimental.pallas import tpu as pltpu
from jax.experimental.pallas import tpu_sc as plsc

N_NODES = 10000
N_EDGES = 320000
D = 128
N_PAD = 10240
E_ROWS = N_EDGES // 128
NC = 2
NS = 16
NW = NC * NS
ROWS_PER_TILE = N_PAD // NS

_sc_mesh = plsc.VectorSubcoreMesh(core_axis_name="c", subcore_axis_name="s")
_sc_params = pltpu.CompilerParams(needs_layout_passes=False)


@functools.partial(
    pl.kernel,
    out_type=jax.ShapeDtypeStruct((2 * NW, N_PAD), jnp.float32),
    mesh=_sc_mesh,
    compiler_params=_sc_params,
    scratch_types=[
        pltpu.VMEM((128,), jnp.int32),
        pltpu.VMEM((128,), jnp.int32),
        pltpu.VMEM((N_PAD,), jnp.float32),
        pltpu.VMEM((N_PAD,), jnp.float32),
    ],
)
def _degree_kernel(src_hbm, dst_hbm, out_hbm, src_v, dst_v, dego_v, degi_v):
    cid = lax.axis_index("c")
    sid = lax.axis_index("s")
    wid = sid * NC + cid

    zeros16 = jnp.zeros((16,), jnp.float32)

    def zero_body(k, _):
        dego_v[pl.ds(k * 16, 16)] = zeros16
        degi_v[pl.ds(k * 16, 16)] = zeros16
        return 0

    lax.fori_loop(0, N_PAD // 16, zero_body, 0)

    ones16 = jnp.ones((16,), jnp.float32)
    n_rows = (E_ROWS - wid + NW - 1) // NW

    def row_body(i, _):
        r = wid + i * NW
        pltpu.sync_copy(src_hbm.at[r], src_v)
        pltpu.sync_copy(dst_hbm.at[r], dst_v)
        for j in range(8):
            s16 = src_v[pl.ds(j * 16, 16)]
            plsc.addupdate_scatter(dego_v, [s16], ones16)
            d16 = dst_v[pl.ds(j * 16, 16)]
            plsc.addupdate_scatter(degi_v, [d16], ones16)
        return 0

    lax.fori_loop(0, n_rows, row_body, 0)

    pltpu.sync_copy(dego_v, out_hbm.at[wid])
    pltpu.sync_copy(degi_v, out_hbm.at[NW + wid])


def _norm_body(deg_ref, out_ref):
    d = deg_ref[...]
    deg_out = jnp.sum(d[:NW, :], axis=0, keepdims=True)
    deg_in = jnp.sum(d[NW:, :], axis=0, keepdims=True)
    s = jnp.concatenate([deg_out, deg_in], axis=0)
    out_ref[...] = lax.rsqrt(jnp.maximum(s, 1.0))


def _norms(deg_partial):
    return pl.pallas_call(
        _norm_body,
        out_shape=jax.ShapeDtypeStruct((2, N_PAD), jnp.float32),
    )(deg_partial)


@functools.partial(
    pl.kernel,
    out_type=(
        jax.ShapeDtypeStruct((N_PAD, D), jnp.float32),
        jax.ShapeDtypeStruct((N_PAD, D), jnp.float32),
    ),
    mesh=_sc_mesh,
    compiler_params=_sc_params,
    scratch_types=[
        pltpu.VMEM((N_PAD,), jnp.float32),
        pltpu.VMEM((128,), jnp.int32),
        pltpu.VMEM((128,), jnp.int32),
        pltpu.VMEM((128,), jnp.float32),
        pltpu.VMEM((128,), jnp.float32),
        pltpu.VMEM((128, D), jnp.float32),
        pltpu.VMEM_SHARED((N_PAD, D), jnp.float32),
        pltpu.SemaphoreType.DMA,
    ],
)
def _agg_kernel(x_hbm, src_hbm, dst_hbm, w_hbm, norm_hbm,
                out0_hbm, out1_hbm, norm_v, src_v, dst_v, w_v, wp_v,
                rows_v, agg_sh, sem):
    cid = lax.axis_index("c")
    sid = lax.axis_index("s")
    wid = sid * NC + cid

    zeros16 = jnp.zeros((16,), jnp.float32)

    def zero_body(k, _):
        rows_v[k // 8, pl.ds((k % 8) * 16, 16)] = zeros16
        return 0

    lax.fori_loop(0, 128 * 8, zero_body, 0)
    for b in range(ROWS_PER_TILE // 128):
        pltpu.sync_copy(rows_v, agg_sh.at[pl.ds(sid * ROWS_PER_TILE + b * 128, 128)])
    pltpu.sync_copy(norm_hbm, norm_v)
    plsc.subcore_barrier()

    n_rows = (E_ROWS - wid + NW - 1) // NW

    def row_body(i, _):
        r = wid + i * NW
        pltpu.sync_copy(src_hbm.at[r], src_v)
        pltpu.sync_copy(dst_hbm.at[r], dst_v)
        pltpu.sync_copy(w_hbm.at[r], w_v)
        pltpu.async_copy(x_hbm.at[src_v], rows_v, sem).wait()
        for j in range(8):
            idx16 = src_v[pl.ds(j * 16, 16)]
            nrm16 = plsc.load_gather(norm_v, [idx16])
            wp_v[pl.ds(j * 16, 16)] = w_v[pl.ds(j * 16, 16)] * nrm16

        def edge_body(e, _):
            wsc = plsc.load_gather(wp_v, [jnp.full((16,), e, jnp.int32)])
            for f in range(8):
                rows_v[e, pl.ds(f * 16, 16)] = rows_v[e, pl.ds(f * 16, 16)] * wsc
            return 0

        lax.fori_loop(0, 128, edge_body, 0, unroll=8)
        pltpu.sync_copy(rows_v, agg_sh.at[dst_v], add=True)
        return 0

    lax.fori_loop(0, n_rows, row_body, 0)
    plsc.subcore_barrier()

    @pl.when(cid == 0)
    def _():
        for b in range(ROWS_PER_TILE // 128):
            off = sid * ROWS_PER_TILE + b * 128
            pltpu.sync_copy(agg_sh.at[pl.ds(off, 128)], out0_hbm.at[pl.ds(off, 128)])

    @pl.when(cid == 1)
    def _():
        for b in range(ROWS_PER_TILE // 128):
            off = sid * ROWS_PER_TILE + b * 128
            pltpu.sync_copy(agg_sh.at[pl.ds(off, 128)], out1_hbm.at[pl.ds(off, 128)])


def _mlp_body(a0_ref, a1_ref, nd_ref, wc_ref, bc_ref, wf_ref, bf_ref,
              w2_ref, b2_ref, out_ref):
    h = (a0_ref[...] + a1_ref[...]) * nd_ref[...]
    h = jnp.dot(h, wc_ref[...], preferred_element_type=jnp.float32) + bc_ref[...]
    h = jnp.maximum(h, 0.0)
    h = jnp.dot(h, wf_ref[...], preferred_element_type=jnp.float32) + bf_ref[...]
    h = jnp.maximum(h, 0.0)
    out_ref[...] = (
        jnp.dot(h, w2_ref[...], preferred_element_type=jnp.float32) + b2_ref[...]
    )


def _mlp(a0, a1, norm_dst, W_conv, b_conv, W_fc, b_fc, W_fc2, b_fc2):
    BR = 1000
    grid = (N_NODES // BR,)
    row_spec = pl.BlockSpec((BR, D), lambda i: (i, 0))
    nd_spec = pl.BlockSpec((BR, 1), lambda i: (i, 0))
    w_spec = pl.BlockSpec((D, D), lambda i: (0, 0))
    b_spec = pl.BlockSpec((1, D), lambda i: (0, 0))
    return pl.pallas_call(
        _mlp_body,
        grid=grid,
        in_specs=[row_spec, row_spec, nd_spec, w_spec, b_spec, w_spec,
                  b_spec, w_spec, b_spec],
        out_specs=row_spec,
        out_shape=jax.ShapeDtypeStruct((N_NODES, D), jnp.float32),
    )(a0, a1, norm_dst, W_conv, b_conv, W_fc, b_fc, W_fc2, b_fc2)


@jax.jit
def kernel(x, edge_index, edge_weight, W_conv, b_conv, W_fc, b_fc, W_fc2,
           b_fc2):
    src2d = edge_index[0].astype(jnp.int32).reshape(E_ROWS, 128)
    dst2d = edge_index[1].astype(jnp.int32).reshape(E_ROWS, 128)
    w2d = edge_weight.reshape(E_ROWS, 128)

    deg_partial = _degree_kernel(src2d, dst2d)
    norms = _norms(deg_partial)
    agg0, agg1 = _agg_kernel(x, src2d, dst2d, w2d, norms[0])
    return _mlp(agg0[:N_NODES], agg1[:N_NODES],
                norms[1, :N_NODES, None], W_conv,
                b_conv.reshape(1, D), W_fc, b_fc.reshape(1, D), W_fc2,
                b_fc2.reshape(1, D))
